# baseline (speedup 1.0000x reference)
import functools

import numpy as np
import jax
import jax.numpy as jnp
from jax import lax
from jax.experimental import pallas as pl
from jax.experimental.pallas import tpu as pltpu
from jax.experimental.pallas import tpu_sc as plsc

_NC = 2
_NS = 16
_NW = _NC * _NS

_CHUNK = 128
_K = 4
_BLK = 2048

_PACK_PERM = np.arange(128).reshape(4, 2, 16).transpose(0, 2, 1).reshape(-1)


def _sc_gather_bf16(table, idx3):
    nw, j_steps, chunk = idx3.shape
    rows_out = nw * j_steps * chunk
    feat = table.shape[1]
    per_w = j_steps * chunk
    mesh = plsc.VectorSubcoreMesh(core_axis_name="c", subcore_axis_name="s")

    @functools.partial(
        pl.kernel,
        mesh=mesh,
        out_type=jax.ShapeDtypeStruct((rows_out, feat // 2), jnp.uint32),
        scratch_types=(
            [pltpu.VMEM((j_steps, chunk), jnp.int32),
             pltpu.VMEM((2 * chunk, feat), jnp.float32),
             pltpu.VMEM((per_w, feat // 2), jnp.uint32)]
            + [pltpu.SemaphoreType.DMA] * 2
            + [pltpu.SemaphoreType.DMA]
        ),
    )
    def gather_k(table_hbm, idx_hbm, out_hbm, idx_v, rows_v, ebf_v, *sems):
        gsems, ssem = sems[:2], sems[2]
        wid = lax.axis_index("s") * _NC + lax.axis_index("c")
        pltpu.sync_copy(idx_hbm.at[wid], idx_v)
        base = wid * per_w

        def fire(j):
            return pltpu.async_copy(
                table_hbm.at[idx_v.at[j]],
                rows_v.at[pl.ds((j % 2) * chunk, chunk)],
                gsems[j % 2],
            )

        gathers = [fire(j) for j in range(min(2, j_steps))]
        half = jnp.uint32(0x8000)
        himask = jnp.uint32(0xFFFF0000)
        scatters = []
        for j in range(j_steps):
            gathers[j].wait()
            slot = (j % 2) * chunk

            def conv_row(r, carry, j=j, slot=slot):
                for g in range(feat // 32):
                    a = rows_v[slot + r, pl.ds(32 * g, 16)]
                    b = rows_v[slot + r, pl.ds(32 * g + 16, 16)]
                    au = lax.bitcast_convert_type(a, jnp.uint32)
                    bu = lax.bitcast_convert_type(b, jnp.uint32)
                    lo = lax.shift_right_logical(au + half, jnp.uint32(16))
                    hi = (bu + half) & himask
                    ebf_v[j * chunk + r, pl.ds(16 * g, 16)] = lo | hi
                return carry

            lax.fori_loop(0, chunk, conv_row, 0)
            if j + 2 < j_steps:
                gathers.append(fire(j + 2))
            scatters.append(
                pltpu.async_copy(
                    ebf_v.at[pl.ds(j * chunk, chunk)],
                    out_hbm.at[pl.ds(base + j * chunk, chunk)],
                    ssem,
                )
            )
        for s in scatters:
            s.wait()

    return gather_k(table, idx3)


def _film_body(e_ref, x_ref, w_ref, b_ref, out_ref):
    feat = x_ref.shape[-1]
    h = jnp.dot(e_ref[...].astype(jnp.float32), w_ref[...],
                preferred_element_type=jnp.float32)
    h = h + b_ref[...]
    out_ref[...] = x_ref[...] * h[:, :feat] + h[:, feat:]


def _film_body_chained(e_ref, x_ref, w_ref, b_ref, buf_ref, out_ref):
    del buf_ref
    _film_body(e_ref, x_ref, w_ref, b_ref, out_ref)


def _film_chunk(e_k, x2, Wp, b2, buf, k, rows, feat):
    chunk_rows = e_k.shape[0]
    nb = chunk_rows // _BLK
    e_spec = pl.BlockSpec((_BLK, feat), lambda i: (i, 0))
    x_spec = pl.BlockSpec((_BLK, feat), lambda i: (k * nb + i, 0))
    w_spec = pl.BlockSpec((feat, 2 * feat), lambda i: (0, 0))
    b_spec = pl.BlockSpec((1, 2 * feat), lambda i: (0, 0))
    out_spec = pl.BlockSpec((_BLK, feat), lambda i: (k * nb + i, 0))
    out_shape = jax.ShapeDtypeStruct((rows, feat), jnp.float32)
    if buf is None:
        return pl.pallas_call(
            _film_body,
            grid=(nb,),
            in_specs=[e_spec, x_spec, w_spec, b_spec],
            out_specs=out_spec,
            out_shape=out_shape,
        )(e_k, x2, Wp, b2)
    buf_spec = pl.BlockSpec((8, feat), lambda i: (0, 0))
    return pl.pallas_call(
        _film_body_chained,
        grid=(nb,),
        in_specs=[e_spec, x_spec, w_spec, b_spec, buf_spec],
        out_specs=out_spec,
        out_shape=out_shape,
        input_output_aliases={4: 0},
    )(e_k, x2, Wp, b2, buf)


def kernel(x, patch_idx, group_idx, embeddings, W, b):
    batch, patch, feat = x.shape
    n_groups, n_nodes, _ = embeddings.shape
    rows = batch * patch

    table = embeddings.reshape(n_groups * n_nodes, feat)
    flat_idx = (group_idx.astype(jnp.int32)[:, None] * n_nodes
                + patch_idx.astype(jnp.int32))
    j_steps = rows // (_K * _NW * _CHUNK)
    idx4 = flat_idx.reshape(_K, _NW, j_steps, _CHUNK)

    e_chunks = [
        lax.bitcast_convert_type(_sc_gather_bf16(table, idx4[k]),
                                 jnp.bfloat16).reshape(rows // _K, feat)
        for k in range(_K)
    ]

    Wp = W[jnp.asarray(_PACK_PERM), :]
    x2 = x.reshape(rows, feat)
    b2 = b.reshape(1, 2 * feat)
    buf = None
    for k in range(_K):
        buf = _film_chunk(e_chunks[k], x2, Wp, b2, buf, k, rows, feat)
    return buf.reshape(batch, patch, feat)


# speedup vs baseline: 1.0028x; 1.0028x over previous
import functools

import numpy as np
import jax
import jax.numpy as jnp
from jax import lax
from jax.experimental import pallas as pl
from jax.experimental.pallas import tpu as pltpu
from jax.experimental.pallas import tpu_sc as plsc

_NC = 2
_NS = 16
_NW = _NC * _NS

_CHUNK = 128
_K = 4
_BLK = 2048

_PACK_PERM = np.arange(128).reshape(4, 2, 16).transpose(0, 2, 1).reshape(-1)


def _sc_gather_bf16(table, idx3):
    nw, j_steps, chunk = idx3.shape
    rows_out = nw * j_steps * chunk
    feat = table.shape[1]
    per_w = j_steps * chunk
    mesh = plsc.VectorSubcoreMesh(core_axis_name="c", subcore_axis_name="s")

    @functools.partial(
        pl.kernel,
        mesh=mesh,
        out_type=jax.ShapeDtypeStruct((rows_out, feat // 2), jnp.uint32),
        scratch_types=(
            [pltpu.VMEM((j_steps, chunk), jnp.int32),
             pltpu.VMEM((2 * chunk, feat), jnp.float32),
             pltpu.VMEM((per_w, feat // 2), jnp.uint32)]
            + [pltpu.SemaphoreType.DMA] * 2
            + [pltpu.SemaphoreType.DMA]
        ),
    )
    def gather_k(table_hbm, idx_hbm, out_hbm, idx_v, rows_v, ebf_v, *sems):
        gsems, ssem = sems[:2], sems[2]
        wid = lax.axis_index("s") * _NC + lax.axis_index("c")
        pltpu.sync_copy(idx_hbm.at[wid], idx_v)
        base = wid * per_w

        def fire(j):
            return pltpu.async_copy(
                table_hbm.at[idx_v.at[j]],
                rows_v.at[pl.ds((j % 2) * chunk, chunk)],
                gsems[j % 2],
            )

        gathers = [fire(j) for j in range(min(2, j_steps))]
        himask = jnp.uint32(0xFFFF0000)
        scatters = []
        for j in range(j_steps):
            gathers[j].wait()
            slot = (j % 2) * chunk

            @plsc.parallel_loop(0, chunk, 1, unroll=4)
            def conv_row(r, j=j, slot=slot):
                for g in range(feat // 32):
                    a = rows_v[slot + r, pl.ds(32 * g, 16)]
                    b = rows_v[slot + r, pl.ds(32 * g + 16, 16)]
                    au = lax.bitcast_convert_type(a, jnp.uint32)
                    bu = lax.bitcast_convert_type(b, jnp.uint32)
                    lo = lax.shift_right_logical(au, jnp.uint32(16))
                    hi = bu & himask
                    ebf_v[j * chunk + r, pl.ds(16 * g, 16)] = lo | hi
            if j + 2 < j_steps:
                gathers.append(fire(j + 2))
            scatters.append(
                pltpu.async_copy(
                    ebf_v.at[pl.ds(j * chunk, chunk)],
                    out_hbm.at[pl.ds(base + j * chunk, chunk)],
                    ssem,
                )
            )
        for s in scatters:
            s.wait()

    return gather_k(table, idx3)


def _film_body(e_ref, x_ref, w_ref, b_ref, out_ref):
    feat = x_ref.shape[-1]
    h = jnp.dot(e_ref[...].astype(jnp.float32), w_ref[...],
                preferred_element_type=jnp.float32)
    h = h + b_ref[...]
    out_ref[...] = x_ref[...] * h[:, :feat] + h[:, feat:]


def _film_body_chained(e_ref, x_ref, w_ref, b_ref, buf_ref, out_ref):
    del buf_ref
    _film_body(e_ref, x_ref, w_ref, b_ref, out_ref)


def _film_chunk(e_k, x2, Wp, b2, buf, k, rows, feat):
    chunk_rows = e_k.shape[0]
    nb = chunk_rows // _BLK
    e_spec = pl.BlockSpec((_BLK, feat), lambda i: (i, 0))
    x_spec = pl.BlockSpec((_BLK, feat), lambda i: (k * nb + i, 0))
    w_spec = pl.BlockSpec((feat, 2 * feat), lambda i: (0, 0))
    b_spec = pl.BlockSpec((1, 2 * feat), lambda i: (0, 0))
    out_spec = pl.BlockSpec((_BLK, feat), lambda i: (k * nb + i, 0))
    out_shape = jax.ShapeDtypeStruct((rows, feat), jnp.float32)
    if buf is None:
        return pl.pallas_call(
            _film_body,
            grid=(nb,),
            in_specs=[e_spec, x_spec, w_spec, b_spec],
            out_specs=out_spec,
            out_shape=out_shape,
        )(e_k, x2, Wp, b2)
    buf_spec = pl.BlockSpec((8, feat), lambda i: (0, 0))
    return pl.pallas_call(
        _film_body_chained,
        grid=(nb,),
        in_specs=[e_spec, x_spec, w_spec, b_spec, buf_spec],
        out_specs=out_spec,
        out_shape=out_shape,
        input_output_aliases={4: 0},
    )(e_k, x2, Wp, b2, buf)


def kernel(x, patch_idx, group_idx, embeddings, W, b):
    batch, patch, feat = x.shape
    n_groups, n_nodes, _ = embeddings.shape
    rows = batch * patch

    table = embeddings.reshape(n_groups * n_nodes, feat)
    flat_idx = (group_idx.astype(jnp.int32)[:, None] * n_nodes
                + patch_idx.astype(jnp.int32))
    j_steps = rows // (_K * _NW * _CHUNK)
    idx4 = flat_idx.reshape(_K, _NW, j_steps, _CHUNK)

    e_chunks = [
        lax.bitcast_convert_type(_sc_gather_bf16(table, idx4[k]),
                                 jnp.bfloat16).reshape(rows // _K, feat)
        for k in range(_K)
    ]

    Wp = W[jnp.asarray(_PACK_PERM), :]
    x2 = x.reshape(rows, feat)
    b2 = b.reshape(1, 2 * feat)
    buf = None
    for k in range(_K):
        buf = _film_chunk(e_chunks[k], x2, Wp, b2, buf, k, rows, feat)
    return buf.reshape(batch, patch, feat)


# speedup vs baseline: 2.2398x; 2.2335x over previous
import functools

import numpy as np
import jax
import jax.numpy as jnp
from jax import lax
from jax.experimental import pallas as pl
from jax.experimental.pallas import tpu as pltpu
from jax.experimental.pallas import tpu_sc as plsc

_NC = 2
_NS = 16
_NW = _NC * _NS

_CHUNK = 128
_K = 4
_BLK = 2048


def _sc_gather_bf16(table, idx3):
    nw, j_steps, chunk = idx3.shape
    rows_out = nw * j_steps * chunk
    feat = table.shape[1]
    per_w = j_steps * chunk
    mesh = plsc.VectorSubcoreMesh(core_axis_name="c", subcore_axis_name="s")

    @functools.partial(
        pl.kernel,
        mesh=mesh,
        out_type=jax.ShapeDtypeStruct((rows_out, feat // 2), jnp.uint32),
        scratch_types=(
            [pltpu.VMEM((j_steps, chunk), jnp.int32),
             pltpu.VMEM((2 * chunk, feat), jnp.float32),
             pltpu.VMEM((per_w, feat // 2), jnp.uint32)]
            + [pltpu.SemaphoreType.DMA] * 2
            + [pltpu.SemaphoreType.DMA]
        ),
    )
    def gather_k(table_hbm, idx_hbm, out_hbm, idx_v, rows_v, ebf_v, *sems):
        gsems, ssem = sems[:2], sems[2]
        wid = lax.axis_index("s") * _NC + lax.axis_index("c")
        pltpu.sync_copy(idx_hbm.at[wid], idx_v)
        base = wid * per_w

        def fire(j):
            return pltpu.async_copy(
                table_hbm.at[idx_v.at[j]],
                rows_v.at[pl.ds((j % 2) * chunk, chunk)],
                gsems[j % 2],
            )

        gathers = [fire(j) for j in range(min(2, j_steps))]
        himask = jnp.uint32(0xFFFF0000)
        scatters = []
        for j in range(j_steps):
            gathers[j].wait()
            slot = (j % 2) * chunk

            @plsc.parallel_loop(0, chunk, 1, unroll=4)
            def conv_row(r, j=j, slot=slot):
                for g in range(feat // 32):
                    a = rows_v[slot + r, pl.ds(32 * g, 16)]
                    b = rows_v[slot + r, pl.ds(32 * g + 16, 16)]
                    au = lax.bitcast_convert_type(a, jnp.uint32)
                    bu = lax.bitcast_convert_type(b, jnp.uint32)
                    lo = lax.shift_right_logical(au, jnp.uint32(16))
                    hi = bu & himask
                    ebf_v[j * chunk + r, pl.ds(16 * g, 16)] = lo | hi
            if j + 2 < j_steps:
                gathers.append(fire(j + 2))
            scatters.append(
                pltpu.async_copy(
                    ebf_v.at[pl.ds(j * chunk, chunk)],
                    out_hbm.at[pl.ds(base + j * chunk, chunk)],
                    ssem,
                )
            )
        for s in scatters:
            s.wait()

    return gather_k(table, idx3)


def _film_body(e_ref, x_ref, wa_ref, wb_ref, b_ref, out_ref):
    feat = x_ref.shape[-1]
    eu = e_ref[...]
    ea = lax.bitcast_convert_type(eu << jnp.uint32(16), jnp.float32)
    eb = lax.bitcast_convert_type(eu & jnp.uint32(0xFFFF0000), jnp.float32)
    h = jnp.dot(ea, wa_ref[...], preferred_element_type=jnp.float32)
    h = h + jnp.dot(eb, wb_ref[...], preferred_element_type=jnp.float32)
    h = h + b_ref[...]
    out_ref[...] = x_ref[...] * h[:, :feat] + h[:, feat:]


def _film_body_chained(e_ref, x_ref, wa_ref, wb_ref, b_ref, buf_ref, out_ref):
    del buf_ref
    _film_body(e_ref, x_ref, wa_ref, wb_ref, b_ref, out_ref)


def _film_chunk(e_k, x2, Wa, Wb, b2, buf, k, rows, feat):
    chunk_rows = e_k.shape[0]
    nb = chunk_rows // _BLK
    e_spec = pl.BlockSpec((_BLK, feat // 2), lambda i: (i, 0))
    x_spec = pl.BlockSpec((_BLK, feat), lambda i: (k * nb + i, 0))
    w_spec = pl.BlockSpec((feat // 2, 2 * feat), lambda i: (0, 0))
    b_spec = pl.BlockSpec((1, 2 * feat), lambda i: (0, 0))
    out_spec = pl.BlockSpec((_BLK, feat), lambda i: (k * nb + i, 0))
    out_shape = jax.ShapeDtypeStruct((rows, feat), jnp.float32)
    if buf is None:
        return pl.pallas_call(
            _film_body,
            grid=(nb,),
            in_specs=[e_spec, x_spec, w_spec, w_spec, b_spec],
            out_specs=out_spec,
            out_shape=out_shape,
        )(e_k, x2, Wa, Wb, b2)
    buf_spec = pl.BlockSpec((8, feat), lambda i: (0, 0))
    return pl.pallas_call(
        _film_body_chained,
        grid=(nb,),
        in_specs=[e_spec, x_spec, w_spec, w_spec, b_spec, buf_spec],
        out_specs=out_spec,
        out_shape=out_shape,
        input_output_aliases={5: 0},
    )(e_k, x2, Wa, Wb, b2, buf)


def kernel(x, patch_idx, group_idx, embeddings, W, b):
    batch, patch, feat = x.shape
    n_groups, n_nodes, _ = embeddings.shape
    rows = batch * patch

    table = embeddings.reshape(n_groups * n_nodes, feat)
    flat_idx = (group_idx.astype(jnp.int32)[:, None] * n_nodes
                + patch_idx.astype(jnp.int32))
    j_steps = rows // (_K * _NW * _CHUNK)
    idx4 = flat_idx.reshape(_K, _NW, j_steps, _CHUNK)

    e_chunks = [_sc_gather_bf16(table, idx4[k]) for k in range(_K)]

    w4 = W.reshape(feat // 32, 2, 16, 2 * feat)
    Wa = w4[:, 0].reshape(feat // 2, 2 * feat)
    Wb = w4[:, 1].reshape(feat // 2, 2 * feat)
    x2 = x.reshape(rows, feat)
    b2 = b.reshape(1, 2 * feat)
    buf = None
    for k in range(_K):
        buf = _film_chunk(e_chunks[k], x2, Wa, Wb, b2, buf, k, rows, feat)
    return buf.reshape(batch, patch, feat)
